# baseline (speedup 1.0000x reference)
import functools

import jax
import jax.numpy as jnp
from jax import lax
from jax.experimental import pallas as pl
from jax.experimental.pallas import tpu as pltpu
from jax.experimental.pallas import tpu_sc as plsc

NUM_CODES = 8192
CODE_DEPTH = 256
D_MODEL = 768
ROWS = 8192
ROW_TILE = 256
NUM_ROW_TILES = ROWS // ROW_TILE
CODE_CHUNKS = 4
CHUNK = NUM_CODES // CODE_CHUNKS


def _tc_body(x_ref, wt_ref, b_ref, cb_ref, c2_ref, col_ref, soft_ref, idx_ref):
    z = jnp.dot(x_ref[...], wt_ref[...], preferred_element_type=jnp.float32)
    z = z + b_ref[...]
    norm = jnp.sqrt(jnp.sum(z * z, axis=1, keepdims=True))
    zn = z / norm
    soft_ref[...] = zn
    znm2 = zn * (-2.0)
    zn2 = jnp.sum(zn * zn, axis=1, keepdims=True)
    m = None
    idxf = None
    for c in range(CODE_CHUNKS):
        sl = pl.ds(c * CHUNK, CHUNK)
        sm2 = lax.dot_general(
            znm2, cb_ref[sl, :],
            dimension_numbers=(((1,), (1,)), ((), ())),
            preferred_element_type=jnp.float32,
        )
        d2 = (zn2 + sm2) + c2_ref[:, sl]
        d = d2 * lax.rsqrt(d2)
        mc = jnp.min(d, axis=1, keepdims=True)
        ic = jnp.min(jnp.where(d == mc, col_ref[:, sl],
                               float(NUM_CODES - 1)), axis=1, keepdims=True)
        if m is None:
            m, idxf = mc, ic
        else:
            idxf = jnp.where(mc < m, ic, idxf)
            m = jnp.minimum(m, mc)
    idx_ref[...] = idxf.astype(jnp.int32).reshape(1, 1, ROW_TILE)


def _tc_quantize(xf, wt, b2, cb, c2, col):
    return pl.pallas_call(
        _tc_body,
        grid=(NUM_ROW_TILES,),
        in_specs=[
            pl.BlockSpec((ROW_TILE, D_MODEL), lambda i: (i, 0)),
            pl.BlockSpec((D_MODEL, CODE_DEPTH), lambda i: (0, 0)),
            pl.BlockSpec((1, CODE_DEPTH), lambda i: (0, 0)),
            pl.BlockSpec((NUM_CODES, CODE_DEPTH), lambda i: (0, 0)),
            pl.BlockSpec((1, NUM_CODES), lambda i: (0, 0)),
            pl.BlockSpec((1, NUM_CODES), lambda i: (0, 0)),
        ],
        out_specs=[
            pl.BlockSpec((ROW_TILE, CODE_DEPTH), lambda i: (i, 0)),
            pl.BlockSpec((1, 1, ROW_TILE), lambda i: (i, 0, 0)),
        ],
        out_shape=[
            jax.ShapeDtypeStruct((ROWS, CODE_DEPTH), jnp.float32),
            jax.ShapeDtypeStruct((NUM_ROW_TILES, 1, ROW_TILE), jnp.int32),
        ],
        compiler_params=pltpu.CompilerParams(
            dimension_semantics=("arbitrary",),
        ),
    )(xf, wt, b2, cb, c2, col)


_NW = 32
_RPW = ROWS // _NW
_CHUNK = 128


def _sc_gather_body(table_hbm, idx_hbm, out_hbm, idx_v, rows_v, sem):
    wid = lax.axis_index("s") * 2 + lax.axis_index("c")
    nchunks = _RPW // _CHUNK
    pltpu.sync_copy(idx_hbm.at[pl.ds(wid * nchunks, nchunks)], idx_v)
    for j in range(nchunks):
        pltpu.async_copy(
            table_hbm.at[idx_v.at[j]],
            rows_v.at[pl.ds(j * _CHUNK, _CHUNK)],
            sem,
        )
    for j in range(nchunks):
        pltpu.make_async_copy(
            table_hbm.at[idx_v.at[j]],
            rows_v.at[pl.ds(j * _CHUNK, _CHUNK)],
            sem,
        ).wait()
    pltpu.sync_copy(rows_v, out_hbm.at[pl.ds(wid * _RPW, _RPW)])


@functools.cache
def _make_sc_gather():
    return pl.kernel(
        _sc_gather_body,
        out_type=jax.ShapeDtypeStruct((ROWS, CODE_DEPTH), jnp.float32),
        mesh=plsc.VectorSubcoreMesh(core_axis_name="c", subcore_axis_name="s"),
        scratch_types=[
            pltpu.VMEM((_RPW // _CHUNK, _CHUNK), jnp.int32),
            pltpu.VMEM((_RPW, CODE_DEPTH), jnp.float32),
            pltpu.SemaphoreType.DMA,
        ],
    )


def kernel(x, W_down, b_down, codebook):
    B, N, _ = x.shape
    xf = x.reshape(ROWS, D_MODEL)
    wt = W_down.T
    b2 = b_down.reshape(1, CODE_DEPTH)
    cb = codebook[0]
    c2 = jnp.sum(cb * cb, axis=1)
    c2 = c2.at[0].add(1e30)
    c2 = c2.reshape(1, NUM_CODES)
    col = jnp.arange(NUM_CODES, dtype=jnp.float32).reshape(1, NUM_CODES)

    soft, idx3 = _tc_quantize(xf, wt, b2, cb, c2, col)
    idx = idx3.reshape(ROWS)

    hard = _make_sc_gather()(cb, idx.reshape(ROWS // _CHUNK, _CHUNK))

    soft_codes = soft.reshape(B, N, CODE_DEPTH)
    code_indices = idx.reshape(B, N)
    hard_codes = hard.reshape(B, N, CODE_DEPTH)
    return (soft_codes, code_indices, hard_codes)


# speedup vs baseline: 1.1415x; 1.1415x over previous
import functools

import jax
import jax.numpy as jnp
from jax import lax
from jax.experimental import pallas as pl
from jax.experimental.pallas import tpu as pltpu
from jax.experimental.pallas import tpu_sc as plsc

NUM_CODES = 8192
CODE_DEPTH = 256
D_MODEL = 768
ROWS = 8192
ROW_TILE = 512
NUM_ROW_TILES = ROWS // ROW_TILE


def _tc_body(x_ref, wt_ref, b_ref, cb_ref, c2_ref, col_ref, soft_ref, idx_ref):
    z = jnp.dot(x_ref[...], wt_ref[...], preferred_element_type=jnp.float32)
    z = z + b_ref[...]
    norm = jnp.sqrt(jnp.sum(z * z, axis=1, keepdims=True))
    zn = z / norm
    soft_ref[...] = zn
    sm2 = lax.dot_general(
        zn * (-2.0), cb_ref[...],
        dimension_numbers=(((1,), (1,)), ((), ())),
        preferred_element_type=jnp.float32,
    )
    zn2 = jnp.sum(zn * zn, axis=1, keepdims=True)
    d2 = (zn2 + sm2) + c2_ref[...]
    d = d2 * lax.rsqrt(d2)
    m = jnp.min(d, axis=1, keepdims=True)
    idxf = jnp.min(jnp.where(d == m, col_ref[...], float(NUM_CODES - 1)),
                   axis=1)
    idx_ref[...] = idxf.astype(jnp.int32).reshape(1, 1, ROW_TILE)


def _tc_quantize(xf, wt, b2, cb, c2, col):
    return pl.pallas_call(
        _tc_body,
        grid=(NUM_ROW_TILES,),
        in_specs=[
            pl.BlockSpec((ROW_TILE, D_MODEL), lambda i: (i, 0)),
            pl.BlockSpec((D_MODEL, CODE_DEPTH), lambda i: (0, 0)),
            pl.BlockSpec((1, CODE_DEPTH), lambda i: (0, 0)),
            pl.BlockSpec((NUM_CODES, CODE_DEPTH), lambda i: (0, 0)),
            pl.BlockSpec((1, NUM_CODES), lambda i: (0, 0)),
            pl.BlockSpec((1, NUM_CODES), lambda i: (0, 0)),
        ],
        out_specs=[
            pl.BlockSpec((ROW_TILE, CODE_DEPTH), lambda i: (i, 0)),
            pl.BlockSpec((1, 1, ROW_TILE), lambda i: (i, 0, 0)),
        ],
        out_shape=[
            jax.ShapeDtypeStruct((ROWS, CODE_DEPTH), jnp.float32),
            jax.ShapeDtypeStruct((NUM_ROW_TILES, 1, ROW_TILE), jnp.int32),
        ],
        compiler_params=pltpu.CompilerParams(
            dimension_semantics=("arbitrary",),
        ),
    )(xf, wt, b2, cb, c2, col)


_NW = 32
_RPW = ROWS // _NW
_CHUNK = 128


def _sc_gather_body(table_hbm, idx_hbm, out_hbm, idx_v, rows_v, sem):
    wid = lax.axis_index("s") * 2 + lax.axis_index("c")
    nchunks = _RPW // _CHUNK
    pltpu.sync_copy(idx_hbm.at[pl.ds(wid * nchunks, nchunks)], idx_v)
    for j in range(nchunks):
        pltpu.async_copy(
            table_hbm.at[idx_v.at[j]],
            rows_v.at[pl.ds(j * _CHUNK, _CHUNK)],
            sem,
        )
    for j in range(nchunks):
        pltpu.make_async_copy(
            table_hbm.at[idx_v.at[j]],
            rows_v.at[pl.ds(j * _CHUNK, _CHUNK)],
            sem,
        ).wait()
    pltpu.sync_copy(rows_v, out_hbm.at[pl.ds(wid * _RPW, _RPW)])


@functools.cache
def _make_sc_gather():
    return pl.kernel(
        _sc_gather_body,
        out_type=jax.ShapeDtypeStruct((ROWS, CODE_DEPTH), jnp.float32),
        mesh=plsc.VectorSubcoreMesh(core_axis_name="c", subcore_axis_name="s"),
        scratch_types=[
            pltpu.VMEM((_RPW // _CHUNK, _CHUNK), jnp.int32),
            pltpu.VMEM((_RPW, CODE_DEPTH), jnp.float32),
            pltpu.SemaphoreType.DMA,
        ],
    )


def kernel(x, W_down, b_down, codebook):
    B, N, _ = x.shape
    xf = x.reshape(ROWS, D_MODEL)
    wt = W_down.T
    b2 = b_down.reshape(1, CODE_DEPTH)
    cb = codebook[0]
    c2 = jnp.sum(cb * cb, axis=1)
    c2 = c2.at[0].add(1e30)
    c2 = c2.reshape(1, NUM_CODES)
    col = jnp.arange(NUM_CODES, dtype=jnp.float32).reshape(1, NUM_CODES)

    soft, idx3 = _tc_quantize(xf, wt, b2, cb, c2, col)
    idx = idx3.reshape(ROWS)

    hard = _make_sc_gather()(cb, idx.reshape(ROWS // _CHUNK, _CHUNK))

    soft_codes = soft.reshape(B, N, CODE_DEPTH)
    code_indices = idx.reshape(B, N)
    hard_codes = hard.reshape(B, N, CODE_DEPTH)
    return (soft_codes, code_indices, hard_codes)


# speedup vs baseline: 1.1930x; 1.0451x over previous
import functools

import jax
import jax.numpy as jnp
from jax import lax
from jax.experimental import pallas as pl
from jax.experimental.pallas import tpu as pltpu
from jax.experimental.pallas import tpu_sc as plsc

NUM_CODES = 8192
CODE_DEPTH = 256
D_MODEL = 768
ROWS = 8192
ROW_TILE = 1024
NUM_ROW_TILES = ROWS // ROW_TILE


def _tc_body(x_ref, wt_ref, b_ref, cb_ref, c2_ref, col_ref, soft_ref, idx_ref):
    z = jnp.dot(x_ref[...], wt_ref[...], preferred_element_type=jnp.float32)
    z = z + b_ref[...]
    norm = jnp.sqrt(jnp.sum(z * z, axis=1, keepdims=True))
    zn = z / norm
    soft_ref[...] = zn
    sm2 = lax.dot_general(
        zn * (-2.0), cb_ref[...],
        dimension_numbers=(((1,), (1,)), ((), ())),
        preferred_element_type=jnp.float32,
    )
    zn2 = jnp.sum(zn * zn, axis=1, keepdims=True)
    d2 = (zn2 + sm2) + c2_ref[...]
    d = d2 * lax.rsqrt(d2)
    m = jnp.min(d, axis=1, keepdims=True)
    idxf = jnp.min(jnp.where(d == m, col_ref[...], float(NUM_CODES - 1)),
                   axis=1)
    idx_ref[...] = idxf.astype(jnp.int32).reshape(1, 1, ROW_TILE)


def _tc_quantize(xf, wt, b2, cb, c2, col):
    return pl.pallas_call(
        _tc_body,
        grid=(NUM_ROW_TILES,),
        in_specs=[
            pl.BlockSpec((ROW_TILE, D_MODEL), lambda i: (i, 0)),
            pl.BlockSpec((D_MODEL, CODE_DEPTH), lambda i: (0, 0)),
            pl.BlockSpec((1, CODE_DEPTH), lambda i: (0, 0)),
            pl.BlockSpec((NUM_CODES, CODE_DEPTH), lambda i: (0, 0)),
            pl.BlockSpec((1, NUM_CODES), lambda i: (0, 0)),
            pl.BlockSpec((1, NUM_CODES), lambda i: (0, 0)),
        ],
        out_specs=[
            pl.BlockSpec((ROW_TILE, CODE_DEPTH), lambda i: (i, 0)),
            pl.BlockSpec((1, 1, ROW_TILE), lambda i: (i, 0, 0)),
        ],
        out_shape=[
            jax.ShapeDtypeStruct((ROWS, CODE_DEPTH), jnp.float32),
            jax.ShapeDtypeStruct((NUM_ROW_TILES, 1, ROW_TILE), jnp.int32),
        ],
        compiler_params=pltpu.CompilerParams(
            dimension_semantics=("arbitrary",),
        ),
    )(xf, wt, b2, cb, c2, col)


_NW = 32
_RPW = ROWS // _NW
_CHUNK = 128


def _sc_gather_body(table_hbm, idx_hbm, out_hbm, idx_v, rows_v, sem):
    wid = lax.axis_index("s") * 2 + lax.axis_index("c")
    nchunks = _RPW // _CHUNK
    pltpu.sync_copy(idx_hbm.at[pl.ds(wid * nchunks, nchunks)], idx_v)
    for j in range(nchunks):
        pltpu.async_copy(
            table_hbm.at[idx_v.at[j]],
            rows_v.at[pl.ds(j * _CHUNK, _CHUNK)],
            sem,
        )
    for j in range(nchunks):
        pltpu.make_async_copy(
            table_hbm.at[idx_v.at[j]],
            rows_v.at[pl.ds(j * _CHUNK, _CHUNK)],
            sem,
        ).wait()
    pltpu.sync_copy(rows_v, out_hbm.at[pl.ds(wid * _RPW, _RPW)])


@functools.cache
def _make_sc_gather():
    return pl.kernel(
        _sc_gather_body,
        out_type=jax.ShapeDtypeStruct((ROWS, CODE_DEPTH), jnp.float32),
        mesh=plsc.VectorSubcoreMesh(core_axis_name="c", subcore_axis_name="s"),
        scratch_types=[
            pltpu.VMEM((_RPW // _CHUNK, _CHUNK), jnp.int32),
            pltpu.VMEM((_RPW, CODE_DEPTH), jnp.float32),
            pltpu.SemaphoreType.DMA,
        ],
    )


def kernel(x, W_down, b_down, codebook):
    B, N, _ = x.shape
    xf = x.reshape(ROWS, D_MODEL)
    wt = W_down.T
    b2 = b_down.reshape(1, CODE_DEPTH)
    cb = codebook[0]
    c2 = jnp.sum(cb * cb, axis=1)
    c2 = c2.at[0].add(1e30)
    c2 = c2.reshape(1, NUM_CODES)
    col = jnp.arange(NUM_CODES, dtype=jnp.float32).reshape(1, NUM_CODES)

    soft, idx3 = _tc_quantize(xf, wt, b2, cb, c2, col)
    idx = idx3.reshape(ROWS)

    hard = _make_sc_gather()(cb, idx.reshape(ROWS // _CHUNK, _CHUNK))

    soft_codes = soft.reshape(B, N, CODE_DEPTH)
    code_indices = idx.reshape(B, N)
    hard_codes = hard.reshape(B, N, CODE_DEPTH)
    return (soft_codes, code_indices, hard_codes)
